# baseline (speedup 1.0000x reference)
import functools
import math

import jax
import jax.numpy as jnp
from jax import lax
from jax.experimental import pallas as pl
from jax.experimental.pallas import tpu as pltpu
from jax.experimental.pallas import tpu_sc as plsc

_LANES = 16
_CHUNK = 128
_NBUF = 8


def _sigmoid(x):
    return 1.0 / (1.0 + jnp.exp(-x))


def _make_deg_kernel(NW, CH, NP, NC, NS):
    n_per_tile = NP // NS

    mesh = plsc.VectorSubcoreMesh(core_axis_name="c", subcore_axis_name="s")

    @functools.partial(
        pl.kernel,
        out_type=jax.ShapeDtypeStruct((NC, NP), jnp.float32),
        mesh=mesh,
        compiler_params=pltpu.CompilerParams(use_tc_tiling_on_sc=False,
                                             needs_layout_passes=False),
        scratch_types=[
            pltpu.VMEM((CH, _CHUNK), jnp.int32),
            pltpu.VMEM((CH, _CHUNK), jnp.float32),
            pltpu.VMEM((n_per_tile,), jnp.float32),
            pltpu.VMEM_SHARED((NP,), jnp.float32),
            pltpu.SemaphoreType.DMA,
        ],
    )
    def deg_kernel(dst_hbm, w_hbm, out_hbm, dst_v, w_v, zbuf, acc_sh, dsem):
        c = lax.axis_index("c")
        s = lax.axis_index("s")
        wid = c * NS + s

        pltpu.sync_copy(dst_hbm.at[wid], dst_v)
        pltpu.sync_copy(w_hbm.at[wid], w_v)

        def zero_body(i, _):
            zbuf[pl.ds(i * _LANES, _LANES)] = jnp.zeros((_LANES,), jnp.float32)
            return 0
        lax.fori_loop(0, n_per_tile // _LANES, zero_body, 0)
        pltpu.sync_copy(zbuf, acc_sh.at[pl.ds(s * n_per_tile, n_per_tile)])
        plsc.subcore_barrier()

        def group_body(j4, _):
            sd = []
            for b in range(_NBUF):
                j = j4 * _NBUF + b
                sd.append(pltpu.async_copy(
                    w_v.at[j], acc_sh.at[dst_v.at[j]], dsem, add=True))
            for b in range(_NBUF):
                sd[b].wait()
            return 0
        lax.fori_loop(0, CH // _NBUF, group_body, 0)

        plsc.subcore_barrier()
        pltpu.sync_copy(acc_sh.at[pl.ds(s * n_per_tile, n_per_tile)],
                        out_hbm.at[c, pl.ds(s * n_per_tile, n_per_tile)])

    return deg_kernel


def _make_msg_kernel(NW, CH, NP, NC, NS, H):
    n_per_tile = NP // NS

    mesh = plsc.VectorSubcoreMesh(core_axis_name="c", subcore_axis_name="s")

    @functools.partial(
        pl.kernel,
        out_type=jax.ShapeDtypeStruct((NC, NP, H), jnp.float32),
        mesh=mesh,
        compiler_params=pltpu.CompilerParams(use_tc_tiling_on_sc=False,
                                             needs_layout_passes=False),
        scratch_types=[
            pltpu.VMEM((CH, _CHUNK), jnp.int32),
            pltpu.VMEM((CH, _CHUNK), jnp.int32),
            pltpu.VMEM((CH, _CHUNK), jnp.float32),
            pltpu.VMEM((_NBUF * _CHUNK, H // 2), jnp.int32),
            pltpu.VMEM((_NBUF * _CHUNK, H), jnp.float32),
            pltpu.VMEM((_CHUNK, H), jnp.float32),
            pltpu.VMEM_SHARED((NP, H // 2), jnp.int32),
            pltpu.VMEM_SHARED((NP, H), jnp.float32),
            pltpu.SemaphoreType.DMA,
            pltpu.SemaphoreType.DMA,
        ],
    )
    def msg_kernel(src_hbm, dst_hbm, w_hbm, y_hbm, out_hbm,
                   src_v, dst_v, w_v, rows_bf, rows_v, zbuf, y_sh, acc_sh,
                   gsem, ssem):
        c = lax.axis_index("c")
        s = lax.axis_index("s")
        wid = c * NS + s

        pltpu.sync_copy(src_hbm.at[wid], src_v)
        pltpu.sync_copy(dst_hbm.at[wid], dst_v)
        pltpu.sync_copy(w_hbm.at[wid], w_v)
        pltpu.sync_copy(y_hbm.at[pl.ds(s * n_per_tile, n_per_tile)],
                        y_sh.at[pl.ds(s * n_per_tile, n_per_tile)])

        def zero_body(i, _):
            zbuf[i, pl.ds(0, _LANES)] = jnp.zeros((_LANES,), jnp.float32)
            zbuf[i, pl.ds(_LANES, _LANES)] = jnp.zeros((_LANES,), jnp.float32)
            return 0
        lax.fori_loop(0, _CHUNK, zero_body, 0)

        def zero_copy(k, _):
            pltpu.sync_copy(
                zbuf, acc_sh.at[pl.ds(s * n_per_tile + k * _CHUNK, _CHUNK)])
            return 0
        lax.fori_loop(0, n_per_tile // _CHUNK, zero_copy, 0)
        plsc.subcore_barrier()

        def group_body(j4, _):
            gd = []
            sd = []
            for b in range(_NBUF):
                gd.append(pltpu.async_copy(
                    y_sh.at[src_v.at[j4 * _NBUF + b]],
                    rows_bf.at[pl.ds(b * _CHUNK, _CHUNK)], gsem))
            for b in range(_NBUF):
                gd[b].wait()
                j = j4 * _NBUF + b

                def scale_body(g, _, b=b, j=j):
                    wvec = w_v[j, pl.ds(g * _LANES, _LANES)]
                    for l in range(_LANES):
                        e = b * _CHUNK + g * _LANES + l
                        u = rows_bf[e, :]
                        lo = plsc.bitcast(u << 16, jnp.float32)
                        hi = plsc.bitcast(u & jnp.int32(-65536), jnp.float32)
                        vb = jnp.full((_LANES,), wvec[l], jnp.float32)
                        rows_v[e, pl.ds(0, _LANES)] = lo * vb
                        rows_v[e, pl.ds(_LANES, _LANES)] = hi * vb
                    return 0
                lax.fori_loop(0, _CHUNK // _LANES, scale_body, 0)
                sd.append(pltpu.async_copy(
                    rows_v.at[pl.ds(b * _CHUNK, _CHUNK)],
                    acc_sh.at[dst_v.at[j]], ssem, add=True))
            for b in range(_NBUF):
                sd[b].wait()
            return 0
        lax.fori_loop(0, CH // _NBUF, group_body, 0)

        plsc.subcore_barrier()
        pltpu.sync_copy(acc_sh.at[pl.ds(s * n_per_tile, n_per_tile)],
                        out_hbm.at[c, pl.ds(s * n_per_tile, n_per_tile)])

    return msg_kernel


def _tc_a_body(N, degp_ref, x_ref, w1_ref, dinv_ref, y1_ref):
    deg = degp_ref[0] + degp_ref[1] + 1.0
    ok = deg > 0.0
    dinv = jnp.where(ok, lax.rsqrt(jnp.where(ok, deg, 1.0)), 0.0)
    dinv_ref[...] = dinv
    dn = dinv[:N]
    xw = jnp.dot(x_ref[...], w1_ref[...], preferred_element_type=jnp.float32)
    y1_ref[...] = xw * dn


def _tc_b_body(N, aggp_ref, y1_ref, dinv_ref, b1_ref, g_ref, be_ref, w2_ref,
               h1_ref, y2_ref):
    agg = aggp_ref[0] + aggp_ref[1]
    dn = dinv_ref[...][:N]
    g1 = (agg[:N] + y1_ref[...]) * dn + b1_ref[...]
    scale = g_ref[...] * lax.rsqrt(jnp.float32(1.0 + 1e-5))
    h1 = jnp.maximum(g1, 0.0) * scale + be_ref[...]
    h1_ref[...] = h1
    y2_ref[...] = jnp.dot(h1, w2_ref[...],
                          preferred_element_type=jnp.float32) * dn


def _tc_c_body(N, aggp_ref, y2_ref, dinv_ref, b2_ref, g_ref, be_ref,
               h1_ref, x_ref,
               Ai, Bi, Ag, Bg, Ao, Bo, bi1_i, bi1_g, bi1_o,
               Ci, Cg, Co, bi2_i, bi2_g, bi2_o,
               lwa, lwb, lwc, linb_ref, out_ref):
    agg = aggp_ref[0] + aggp_ref[1]
    dn = dinv_ref[...][:N]
    g2 = (agg[:N] + y2_ref[...]) * dn + b2_ref[...]
    scale = g_ref[...] * lax.rsqrt(jnp.float32(1.0 + 1e-5))
    h2 = jnp.maximum(g2, 0.0) * scale + be_ref[...]
    h1 = h1_ref[...]

    def mm(a, b):
        return jnp.dot(a, b, preferred_element_type=jnp.float32)

    gi = _sigmoid(mm(h1, Ai[...]) + mm(h2, Bi[...]) + bi1_i[...])
    gg = jnp.tanh(mm(h1, Ag[...]) + mm(h2, Bg[...]) + bi1_g[...])
    go = _sigmoid(mm(h1, Ao[...]) + mm(h2, Bo[...]) + bi1_o[...])
    hn1 = go * jnp.tanh(gi * gg)

    gi2 = _sigmoid(mm(hn1, Ci[...]) + bi2_i[...])
    gg2 = jnp.tanh(mm(hn1, Cg[...]) + bi2_g[...])
    go2 = _sigmoid(mm(hn1, Co[...]) + bi2_o[...])
    hn2 = go2 * jnp.tanh(gi2 * gg2)

    out = (mm(jnp.maximum(hn1, 0.0), lwa[...])
           + mm(jnp.maximum(hn2, 0.0), lwb[...])
           + mm(jnp.maximum(x_ref[...], 0.0), lwc[...])
           + linb_ref[...])
    out_ref[...] = out


def kernel(x, edge_index, edge_weight, W1, b1, W2, b2, bn1_g, bn1_b,
           bn2_g, bn2_b, l1_wih, l1_whh, l1_bih, l1_bhh,
           l2_wih, l2_whh, l2_bih, l2_bhh, lin_W, lin_b):
    N, F = x.shape
    E = edge_index.shape[1]
    H = W1.shape[1]

    info = plsc.get_sparse_core_info()
    NC, NS = info.num_cores, info.num_subcores
    NW = NC * NS

    CH = math.ceil(E / (NW * _CHUNK))
    CH = ((CH + _NBUF - 1) // _NBUF) * _NBUF
    EP = NW * CH * _CHUNK
    NP = ((N + NS * 8 - 1) // (NS * 8)) * (NS * 8)
    NP = max(NP, _CHUNK * NS)
    npt = NP // NS
    if npt % _CHUNK:
        NP = ((npt + _CHUNK - 1) // _CHUNK) * _CHUNK * NS

    pad = EP - E
    src = jnp.concatenate(
        [edge_index[0], jnp.zeros((pad,), jnp.int32)]).reshape(NW, CH, _CHUNK)
    dst = jnp.concatenate(
        [edge_index[1], jnp.zeros((pad,), jnp.int32)]).reshape(NW, CH, _CHUNK)
    wpad = jnp.concatenate(
        [edge_weight, jnp.zeros((pad,), jnp.float32)]).reshape(NW, CH, _CHUNK)

    deg_kernel = _make_deg_kernel(NW, CH, NP, NC, NS)
    msg_kernel = _make_msg_kernel(NW, CH, NP, NC, NS, H)

    degp = deg_kernel(dst, wpad)
    degp3 = degp.reshape(NC, NP, 1)

    dinv, y1 = pl.pallas_call(
        functools.partial(_tc_a_body, N),
        out_shape=(jax.ShapeDtypeStruct((NP, 1), jnp.float32),
                   jax.ShapeDtypeStruct((N, H), jnp.float32)),
    )(degp3, x, W1)

    ypad = jnp.zeros((NP - N, H // 2), jnp.int32)

    def to_table(y):
        yi = jnp.stack([y[:, :H // 2], y[:, H // 2:]],
                       axis=-1).astype(jnp.bfloat16)
        yu = lax.bitcast_convert_type(yi, jnp.int32)
        return jnp.concatenate([yu, ypad])

    agg1 = msg_kernel(src, dst, wpad, to_table(y1))

    h1, y2 = pl.pallas_call(
        functools.partial(_tc_b_body, N),
        out_shape=(jax.ShapeDtypeStruct((N, H), jnp.float32),
                   jax.ShapeDtypeStruct((N, H), jnp.float32)),
    )(agg1, y1, dinv, b1, bn1_g, bn1_b, W2)

    agg2 = msg_kernel(src, dst, wpad, to_table(y2))

    w1t = l1_wih.T
    b1s = l1_bih + l1_bhh
    Ai, Bi = w1t[:H, 0 * H:1 * H], w1t[H:, 0 * H:1 * H]
    Ag, Bg = w1t[:H, 2 * H:3 * H], w1t[H:, 2 * H:3 * H]
    Ao, Bo = w1t[:H, 3 * H:4 * H], w1t[H:, 3 * H:4 * H]
    w2t = l2_wih.T
    b2s = l2_bih + l2_bhh
    Ci, Cg, Co = w2t[:, 0 * H:1 * H], w2t[:, 2 * H:3 * H], w2t[:, 3 * H:4 * H]
    lwt = lin_W.T
    lwa, lwb, lwc = lwt[:H], lwt[H:2 * H], lwt[2 * H:]

    out = pl.pallas_call(
        functools.partial(_tc_c_body, N),
        out_shape=jax.ShapeDtypeStruct((N, 1), jnp.float32),
    )(agg2, y2, dinv, b2, bn2_g, bn2_b, h1, x,
      Ai, Bi, Ag, Bg, Ao, Bo,
      b1s[0 * H:1 * H], b1s[2 * H:3 * H], b1s[3 * H:4 * H],
      Ci, Cg, Co,
      b2s[0 * H:1 * H], b2s[2 * H:3 * H], b2s[3 * H:4 * H],
      lwa, lwb, lwc, lin_b)
    return out


# speedup vs baseline: 1.3046x; 1.3046x over previous
import functools
import math

import jax
import jax.numpy as jnp
from jax import lax
from jax.experimental import pallas as pl
from jax.experimental.pallas import tpu as pltpu
from jax.experimental.pallas import tpu_sc as plsc

_LANES = 16
_CHUNK = 128
_NBUF = 8


def _sigmoid(x):
    return 1.0 / (1.0 + jnp.exp(-x))


def _make_deg_kernel(NW, CH, NP, NC, NS):
    n_per_tile = NP // NS

    mesh = plsc.VectorSubcoreMesh(core_axis_name="c", subcore_axis_name="s")

    @functools.partial(
        pl.kernel,
        out_type=jax.ShapeDtypeStruct((NC, NP), jnp.float32),
        mesh=mesh,
        compiler_params=pltpu.CompilerParams(use_tc_tiling_on_sc=False),
        scratch_types=[
            pltpu.VMEM((CH, _CHUNK), jnp.int32),
            pltpu.VMEM((CH, _CHUNK), jnp.float32),
            pltpu.VMEM((n_per_tile,), jnp.float32),
            pltpu.VMEM_SHARED((NP,), jnp.float32),
            pltpu.SemaphoreType.DMA,
        ],
    )
    def deg_kernel(dst_hbm, w_hbm, out_hbm, dst_v, w_v, zbuf, acc_sh, dsem):
        c = lax.axis_index("c")
        s = lax.axis_index("s")
        wid = c * NS + s

        pltpu.sync_copy(dst_hbm.at[wid], dst_v)
        pltpu.sync_copy(w_hbm.at[wid], w_v)

        def zero_body(i, _):
            zbuf[pl.ds(i * _LANES, _LANES)] = jnp.zeros((_LANES,), jnp.float32)
            return 0
        lax.fori_loop(0, n_per_tile // _LANES, zero_body, 0)
        pltpu.sync_copy(zbuf, acc_sh.at[pl.ds(s * n_per_tile, n_per_tile)])
        plsc.subcore_barrier()

        def group_body(j4, _):
            sd = []
            for b in range(_NBUF):
                j = j4 * _NBUF + b
                sd.append(pltpu.async_copy(
                    w_v.at[j], acc_sh.at[dst_v.at[j]], dsem, add=True))
            for b in range(_NBUF):
                sd[b].wait()
            return 0
        lax.fori_loop(0, CH // _NBUF, group_body, 0)

        plsc.subcore_barrier()
        pltpu.sync_copy(acc_sh.at[pl.ds(s * n_per_tile, n_per_tile)],
                        out_hbm.at[c, pl.ds(s * n_per_tile, n_per_tile)])

    return deg_kernel


def _make_msg_kernel(NW, CH, NP, NC, NS, H):
    n_per_tile = NP // NS

    mesh = plsc.VectorSubcoreMesh(core_axis_name="c", subcore_axis_name="s")

    @functools.partial(
        pl.kernel,
        out_type=jax.ShapeDtypeStruct((NC, NP, H), jnp.float32),
        mesh=mesh,
        compiler_params=pltpu.CompilerParams(use_tc_tiling_on_sc=False),
        scratch_types=[
            pltpu.VMEM((CH, _CHUNK), jnp.int32),
            pltpu.VMEM((CH, _CHUNK), jnp.int32),
            pltpu.VMEM((CH, _CHUNK), jnp.float32),
            pltpu.VMEM((_NBUF * _CHUNK, H), jnp.float32),
            pltpu.VMEM((_CHUNK, H), jnp.float32),
            pltpu.VMEM_SHARED((NP, H), jnp.float32),
            pltpu.VMEM_SHARED((NP, H), jnp.float32),
            pltpu.SemaphoreType.DMA,
            pltpu.SemaphoreType.DMA,
        ],
    )
    def msg_kernel(src_hbm, dst_hbm, w_hbm, y_hbm, out_hbm,
                   src_v, dst_v, w_v, rows_v, zbuf, y_sh, acc_sh,
                   gsem, ssem):
        c = lax.axis_index("c")
        s = lax.axis_index("s")
        wid = c * NS + s

        pltpu.sync_copy(src_hbm.at[wid], src_v)
        pltpu.sync_copy(dst_hbm.at[wid], dst_v)
        pltpu.sync_copy(w_hbm.at[wid], w_v)
        pltpu.sync_copy(y_hbm.at[pl.ds(s * n_per_tile, n_per_tile)],
                        y_sh.at[pl.ds(s * n_per_tile, n_per_tile)])

        def zero_body(i, _):
            zbuf[i, pl.ds(0, _LANES)] = jnp.zeros((_LANES,), jnp.float32)
            zbuf[i, pl.ds(_LANES, _LANES)] = jnp.zeros((_LANES,), jnp.float32)
            return 0
        lax.fori_loop(0, _CHUNK, zero_body, 0)

        def zero_copy(k, _):
            pltpu.sync_copy(
                zbuf, acc_sh.at[pl.ds(s * n_per_tile + k * _CHUNK, _CHUNK)])
            return 0
        lax.fori_loop(0, n_per_tile // _CHUNK, zero_copy, 0)
        plsc.subcore_barrier()

        def group_body(j4, _):
            gd = []
            sd = []
            for b in range(_NBUF):
                gd.append(pltpu.async_copy(
                    y_sh.at[src_v.at[j4 * _NBUF + b]],
                    rows_v.at[pl.ds(b * _CHUNK, _CHUNK)], gsem))
            for b in range(_NBUF):
                gd[b].wait()
                j = j4 * _NBUF + b

                def scale_body(g, _, b=b, j=j):
                    wvec = w_v[j, pl.ds(g * _LANES, _LANES)]
                    for l in range(_LANES):
                        e = b * _CHUNK + g * _LANES + l
                        vb = jnp.full((_LANES,), wvec[l], jnp.float32)
                        rows_v[e, pl.ds(0, _LANES)] = (
                            rows_v[e, pl.ds(0, _LANES)] * vb)
                        rows_v[e, pl.ds(_LANES, _LANES)] = (
                            rows_v[e, pl.ds(_LANES, _LANES)] * vb)
                    return 0
                lax.fori_loop(0, _CHUNK // _LANES, scale_body, 0)
                sd.append(pltpu.async_copy(
                    rows_v.at[pl.ds(b * _CHUNK, _CHUNK)],
                    acc_sh.at[dst_v.at[j]], ssem, add=True))
            for b in range(_NBUF):
                sd[b].wait()
            return 0
        lax.fori_loop(0, CH // _NBUF, group_body, 0)

        plsc.subcore_barrier()
        pltpu.sync_copy(acc_sh.at[pl.ds(s * n_per_tile, n_per_tile)],
                        out_hbm.at[c, pl.ds(s * n_per_tile, n_per_tile)])

    return msg_kernel


def _tc_a_body(N, degp_ref, x_ref, w1_ref, dinv_ref, y1_ref):
    deg = degp_ref[0] + degp_ref[1] + 1.0
    ok = deg > 0.0
    dinv = jnp.where(ok, lax.rsqrt(jnp.where(ok, deg, 1.0)), 0.0)
    dinv_ref[...] = dinv
    dn = dinv[:N]
    xw = jnp.dot(x_ref[...], w1_ref[...], preferred_element_type=jnp.float32)
    y1_ref[...] = xw * dn


def _tc_b_body(N, aggp_ref, y1_ref, dinv_ref, b1_ref, g_ref, be_ref, w2_ref,
               h1_ref, y2_ref):
    agg = aggp_ref[0] + aggp_ref[1]
    dn = dinv_ref[...][:N]
    g1 = (agg[:N] + y1_ref[...]) * dn + b1_ref[...]
    scale = g_ref[...] * lax.rsqrt(jnp.float32(1.0 + 1e-5))
    h1 = jnp.maximum(g1, 0.0) * scale + be_ref[...]
    h1_ref[...] = h1
    y2_ref[...] = jnp.dot(h1, w2_ref[...],
                          preferred_element_type=jnp.float32) * dn


def _tc_c_body(N, aggp_ref, y2_ref, dinv_ref, b2_ref, g_ref, be_ref,
               h1_ref, x_ref,
               Ai, Bi, Ag, Bg, Ao, Bo, bi1_i, bi1_g, bi1_o,
               Ci, Cg, Co, bi2_i, bi2_g, bi2_o,
               lwa, lwb, lwc, linb_ref, out_ref):
    agg = aggp_ref[0] + aggp_ref[1]
    dn = dinv_ref[...][:N]
    g2 = (agg[:N] + y2_ref[...]) * dn + b2_ref[...]
    scale = g_ref[...] * lax.rsqrt(jnp.float32(1.0 + 1e-5))
    h2 = jnp.maximum(g2, 0.0) * scale + be_ref[...]
    h1 = h1_ref[...]

    def mm(a, b):
        return jnp.dot(a, b, preferred_element_type=jnp.float32)

    gi = _sigmoid(mm(h1, Ai[...]) + mm(h2, Bi[...]) + bi1_i[...])
    gg = jnp.tanh(mm(h1, Ag[...]) + mm(h2, Bg[...]) + bi1_g[...])
    go = _sigmoid(mm(h1, Ao[...]) + mm(h2, Bo[...]) + bi1_o[...])
    hn1 = go * jnp.tanh(gi * gg)

    gi2 = _sigmoid(mm(hn1, Ci[...]) + bi2_i[...])
    gg2 = jnp.tanh(mm(hn1, Cg[...]) + bi2_g[...])
    go2 = _sigmoid(mm(hn1, Co[...]) + bi2_o[...])
    hn2 = go2 * jnp.tanh(gi2 * gg2)

    out = (mm(jnp.maximum(hn1, 0.0), lwa[...])
           + mm(jnp.maximum(hn2, 0.0), lwb[...])
           + mm(jnp.maximum(x_ref[...], 0.0), lwc[...])
           + linb_ref[...])
    out_ref[...] = out


def kernel(x, edge_index, edge_weight, W1, b1, W2, b2, bn1_g, bn1_b,
           bn2_g, bn2_b, l1_wih, l1_whh, l1_bih, l1_bhh,
           l2_wih, l2_whh, l2_bih, l2_bhh, lin_W, lin_b):
    N, F = x.shape
    E = edge_index.shape[1]
    H = W1.shape[1]

    info = plsc.get_sparse_core_info()
    NC, NS = info.num_cores, info.num_subcores
    NW = NC * NS

    CH = math.ceil(E / (NW * _CHUNK))
    CH = ((CH + _NBUF - 1) // _NBUF) * _NBUF
    EP = NW * CH * _CHUNK
    NP = ((N + NS * 8 - 1) // (NS * 8)) * (NS * 8)
    NP = max(NP, _CHUNK * NS)
    npt = NP // NS
    if npt % _CHUNK:
        NP = ((npt + _CHUNK - 1) // _CHUNK) * _CHUNK * NS

    pad = EP - E
    src = jnp.concatenate(
        [edge_index[0], jnp.zeros((pad,), jnp.int32)]).reshape(NW, CH, _CHUNK)
    dst = jnp.concatenate(
        [edge_index[1], jnp.zeros((pad,), jnp.int32)]).reshape(NW, CH, _CHUNK)
    wpad = jnp.concatenate(
        [edge_weight, jnp.zeros((pad,), jnp.float32)]).reshape(NW, CH, _CHUNK)

    deg_kernel = _make_deg_kernel(NW, CH, NP, NC, NS)
    msg_kernel = _make_msg_kernel(NW, CH, NP, NC, NS, H)

    degp = deg_kernel(dst, wpad)
    degp3 = degp.reshape(NC, NP, 1)

    dinv, y1 = pl.pallas_call(
        functools.partial(_tc_a_body, N),
        out_shape=(jax.ShapeDtypeStruct((NP, 1), jnp.float32),
                   jax.ShapeDtypeStruct((N, H), jnp.float32)),
    )(degp3, x, W1)

    ypad = jnp.zeros((NP - N, H), jnp.float32)

    def to_table(y):
        return jnp.concatenate([y, ypad])

    agg1 = msg_kernel(src, dst, wpad, to_table(y1))

    h1, y2 = pl.pallas_call(
        functools.partial(_tc_b_body, N),
        out_shape=(jax.ShapeDtypeStruct((N, H), jnp.float32),
                   jax.ShapeDtypeStruct((N, H), jnp.float32)),
    )(agg1, y1, dinv, b1, bn1_g, bn1_b, W2)

    agg2 = msg_kernel(src, dst, wpad, to_table(y2))

    w1t = l1_wih.T
    b1s = l1_bih + l1_bhh
    Ai, Bi = w1t[:H, 0 * H:1 * H], w1t[H:, 0 * H:1 * H]
    Ag, Bg = w1t[:H, 2 * H:3 * H], w1t[H:, 2 * H:3 * H]
    Ao, Bo = w1t[:H, 3 * H:4 * H], w1t[H:, 3 * H:4 * H]
    w2t = l2_wih.T
    b2s = l2_bih + l2_bhh
    Ci, Cg, Co = w2t[:, 0 * H:1 * H], w2t[:, 2 * H:3 * H], w2t[:, 3 * H:4 * H]
    lwt = lin_W.T
    lwa, lwb, lwc = lwt[:H], lwt[H:2 * H], lwt[2 * H:]

    out = pl.pallas_call(
        functools.partial(_tc_c_body, N),
        out_shape=jax.ShapeDtypeStruct((N, 1), jnp.float32),
    )(agg2, y2, dinv, b2, bn2_g, bn2_b, h1, x,
      Ai, Bi, Ag, Bg, Ao, Bo,
      b1s[0 * H:1 * H], b1s[2 * H:3 * H], b1s[3 * H:4 * H],
      Ci, Cg, Co,
      b2s[0 * H:1 * H], b2s[2 * H:3 * H], b2s[3 * H:4 * H],
      lwa, lwb, lwc, lin_b)
    return out


# speedup vs baseline: 1.3190x; 1.0111x over previous
import functools
import math

import jax
import jax.numpy as jnp
from jax import lax
from jax.experimental import pallas as pl
from jax.experimental.pallas import tpu as pltpu
from jax.experimental.pallas import tpu_sc as plsc

_LANES = 16
_CHUNK = 128
_NBUF = 8


def _sigmoid(x):
    return 1.0 / (1.0 + jnp.exp(-x))


def _make_deg_kernel(NW, CH, NP, NC, NS):
    n_per_tile = NP // NS

    mesh = plsc.VectorSubcoreMesh(core_axis_name="c", subcore_axis_name="s")

    @functools.partial(
        pl.kernel,
        out_type=jax.ShapeDtypeStruct((NC, NP), jnp.float32),
        mesh=mesh,
        compiler_params=pltpu.CompilerParams(use_tc_tiling_on_sc=False),
        scratch_types=[
            pltpu.VMEM((CH, _CHUNK), jnp.int32),
            pltpu.VMEM((CH, _CHUNK), jnp.float32),
            pltpu.VMEM((n_per_tile,), jnp.float32),
            pltpu.VMEM_SHARED((NP,), jnp.float32),
            pltpu.SemaphoreType.DMA,
        ],
    )
    def deg_kernel(dst_hbm, w_hbm, out_hbm, dst_v, w_v, zbuf, acc_sh, dsem):
        c = lax.axis_index("c")
        s = lax.axis_index("s")
        wid = c * NS + s

        pltpu.sync_copy(dst_hbm.at[wid], dst_v)
        pltpu.sync_copy(w_hbm.at[wid], w_v)

        def zero_body(i, _):
            zbuf[pl.ds(i * _LANES, _LANES)] = jnp.zeros((_LANES,), jnp.float32)
            return 0
        lax.fori_loop(0, n_per_tile // _LANES, zero_body, 0)
        pltpu.sync_copy(zbuf, acc_sh.at[pl.ds(s * n_per_tile, n_per_tile)])
        plsc.subcore_barrier()

        def group_body(j4, _):
            sd = []
            for b in range(_NBUF):
                j = j4 * _NBUF + b
                sd.append(pltpu.async_copy(
                    w_v.at[j], acc_sh.at[dst_v.at[j]], dsem, add=True))
            for b in range(_NBUF):
                sd[b].wait()
            return 0
        lax.fori_loop(0, CH // _NBUF, group_body, 0)

        plsc.subcore_barrier()
        pltpu.sync_copy(acc_sh.at[pl.ds(s * n_per_tile, n_per_tile)],
                        out_hbm.at[c, pl.ds(s * n_per_tile, n_per_tile)])

    return deg_kernel


def _make_msg_kernel(NW, CH, NP, NC, NS, H):
    n_per_tile = NP // NS

    mesh = plsc.VectorSubcoreMesh(core_axis_name="c", subcore_axis_name="s")

    @functools.partial(
        pl.kernel,
        out_type=jax.ShapeDtypeStruct((NC, NP, H), jnp.float32),
        mesh=mesh,
        compiler_params=pltpu.CompilerParams(use_tc_tiling_on_sc=False),
        scratch_types=[
            pltpu.VMEM((CH, _CHUNK), jnp.int32),
            pltpu.VMEM((CH, _CHUNK), jnp.int32),
            pltpu.VMEM((CH, _CHUNK), jnp.float32),
            pltpu.VMEM((_NBUF * _CHUNK, H), jnp.float32),
            pltpu.VMEM((_CHUNK, H), jnp.float32),
            pltpu.VMEM_SHARED((NP, H), jnp.float32),
            pltpu.VMEM_SHARED((NP, H), jnp.float32),
            pltpu.SemaphoreType.DMA,
            pltpu.SemaphoreType.DMA,
        ],
    )
    def msg_kernel(src_hbm, dst_hbm, w_hbm, y_hbm, out_hbm,
                   src_v, dst_v, w_v, rows_v, zbuf, y_sh, acc_sh,
                   gsem, ssem):
        c = lax.axis_index("c")
        s = lax.axis_index("s")
        wid = c * NS + s

        pltpu.sync_copy(src_hbm.at[wid], src_v)
        pltpu.sync_copy(dst_hbm.at[wid], dst_v)
        pltpu.sync_copy(w_hbm.at[wid], w_v)
        pltpu.sync_copy(y_hbm.at[pl.ds(s * n_per_tile, n_per_tile)],
                        y_sh.at[pl.ds(s * n_per_tile, n_per_tile)])

        def zero_body(i, _):
            zbuf[i, pl.ds(0, _LANES)] = jnp.zeros((_LANES,), jnp.float32)
            zbuf[i, pl.ds(_LANES, _LANES)] = jnp.zeros((_LANES,), jnp.float32)
            return 0
        lax.fori_loop(0, _CHUNK, zero_body, 0)

        def zero_copy(k, _):
            pltpu.sync_copy(
                zbuf, acc_sh.at[pl.ds(s * n_per_tile + k * _CHUNK, _CHUNK)])
            return 0
        lax.fori_loop(0, n_per_tile // _CHUNK, zero_copy, 0)
        plsc.subcore_barrier()

        NG = _NBUF // 2

        def drain_scatters():
            for _ in range(NG):
                pltpu.make_async_copy(
                    rows_v.at[pl.ds(0, _CHUNK)],
                    acc_sh.at[dst_v.at[0]], ssem).wait()

        def group_body(j4, _):
            half = (j4 % 2) * (NG * _CHUNK)
            gd = []
            for b in range(NG):
                gd.append(pltpu.async_copy(
                    y_sh.at[src_v.at[j4 * NG + b]],
                    rows_v.at[pl.ds(half + b * _CHUNK, _CHUNK)], gsem))

            @pl.when(j4 > 0)
            def _():
                drain_scatters()
            for b in range(NG):
                gd[b].wait()
                j = j4 * NG + b

                def scale_body(g, _, b=b, j=j):
                    wvec = w_v[j, pl.ds(g * _LANES, _LANES)]
                    for l in range(_LANES):
                        e = g * _LANES + l
                        vb = jnp.full((_LANES,), wvec[l], jnp.float32)
                        rows_v[half + b * _CHUNK + e, pl.ds(0, _LANES)] = (
                            rows_v[half + b * _CHUNK + e,
                                   pl.ds(0, _LANES)] * vb)
                        rows_v[half + b * _CHUNK + e,
                               pl.ds(_LANES, _LANES)] = (
                            rows_v[half + b * _CHUNK + e,
                                   pl.ds(_LANES, _LANES)] * vb)
                    return 0
                lax.fori_loop(0, _CHUNK // _LANES, scale_body, 0)
                pltpu.async_copy(
                    rows_v.at[pl.ds(half + b * _CHUNK, _CHUNK)],
                    acc_sh.at[dst_v.at[j]], ssem, add=True)
            return 0
        lax.fori_loop(0, CH // NG, group_body, 0)
        drain_scatters()

        plsc.subcore_barrier()
        pltpu.sync_copy(acc_sh.at[pl.ds(s * n_per_tile, n_per_tile)],
                        out_hbm.at[c, pl.ds(s * n_per_tile, n_per_tile)])

    return msg_kernel


def _tc_a_body(N, degp_ref, x_ref, w1_ref, dinv_ref, y1_ref):
    deg = degp_ref[0] + degp_ref[1] + 1.0
    ok = deg > 0.0
    dinv = jnp.where(ok, lax.rsqrt(jnp.where(ok, deg, 1.0)), 0.0)
    dinv_ref[...] = dinv
    dn = dinv[:N]
    xw = jnp.dot(x_ref[...], w1_ref[...], preferred_element_type=jnp.float32)
    y1_ref[...] = xw * dn


def _tc_b_body(N, aggp_ref, y1_ref, dinv_ref, b1_ref, g_ref, be_ref, w2_ref,
               h1_ref, y2_ref):
    agg = aggp_ref[0] + aggp_ref[1]
    dn = dinv_ref[...][:N]
    g1 = (agg[:N] + y1_ref[...]) * dn + b1_ref[...]
    scale = g_ref[...] * lax.rsqrt(jnp.float32(1.0 + 1e-5))
    h1 = jnp.maximum(g1, 0.0) * scale + be_ref[...]
    h1_ref[...] = h1
    y2_ref[...] = jnp.dot(h1, w2_ref[...],
                          preferred_element_type=jnp.float32) * dn


def _tc_c_body(N, aggp_ref, y2_ref, dinv_ref, b2_ref, g_ref, be_ref,
               h1_ref, x_ref,
               Ai, Bi, Ag, Bg, Ao, Bo, bi1_i, bi1_g, bi1_o,
               Ci, Cg, Co, bi2_i, bi2_g, bi2_o,
               lwa, lwb, lwc, linb_ref, out_ref):
    agg = aggp_ref[0] + aggp_ref[1]
    dn = dinv_ref[...][:N]
    g2 = (agg[:N] + y2_ref[...]) * dn + b2_ref[...]
    scale = g_ref[...] * lax.rsqrt(jnp.float32(1.0 + 1e-5))
    h2 = jnp.maximum(g2, 0.0) * scale + be_ref[...]
    h1 = h1_ref[...]

    def mm(a, b):
        return jnp.dot(a, b, preferred_element_type=jnp.float32)

    gi = _sigmoid(mm(h1, Ai[...]) + mm(h2, Bi[...]) + bi1_i[...])
    gg = jnp.tanh(mm(h1, Ag[...]) + mm(h2, Bg[...]) + bi1_g[...])
    go = _sigmoid(mm(h1, Ao[...]) + mm(h2, Bo[...]) + bi1_o[...])
    hn1 = go * jnp.tanh(gi * gg)

    gi2 = _sigmoid(mm(hn1, Ci[...]) + bi2_i[...])
    gg2 = jnp.tanh(mm(hn1, Cg[...]) + bi2_g[...])
    go2 = _sigmoid(mm(hn1, Co[...]) + bi2_o[...])
    hn2 = go2 * jnp.tanh(gi2 * gg2)

    out = (mm(jnp.maximum(hn1, 0.0), lwa[...])
           + mm(jnp.maximum(hn2, 0.0), lwb[...])
           + mm(jnp.maximum(x_ref[...], 0.0), lwc[...])
           + linb_ref[...])
    out_ref[...] = out


def kernel(x, edge_index, edge_weight, W1, b1, W2, b2, bn1_g, bn1_b,
           bn2_g, bn2_b, l1_wih, l1_whh, l1_bih, l1_bhh,
           l2_wih, l2_whh, l2_bih, l2_bhh, lin_W, lin_b):
    N, F = x.shape
    E = edge_index.shape[1]
    H = W1.shape[1]

    info = plsc.get_sparse_core_info()
    NC, NS = info.num_cores, info.num_subcores
    NW = NC * NS

    CH = math.ceil(E / (NW * _CHUNK))
    CH = ((CH + _NBUF - 1) // _NBUF) * _NBUF
    EP = NW * CH * _CHUNK
    NP = ((N + NS * 8 - 1) // (NS * 8)) * (NS * 8)
    NP = max(NP, _CHUNK * NS)
    npt = NP // NS
    if npt % _CHUNK:
        NP = ((npt + _CHUNK - 1) // _CHUNK) * _CHUNK * NS

    pad = EP - E
    src = jnp.concatenate(
        [edge_index[0], jnp.zeros((pad,), jnp.int32)]).reshape(NW, CH, _CHUNK)
    dst = jnp.concatenate(
        [edge_index[1], jnp.zeros((pad,), jnp.int32)]).reshape(NW, CH, _CHUNK)
    wpad = jnp.concatenate(
        [edge_weight, jnp.zeros((pad,), jnp.float32)]).reshape(NW, CH, _CHUNK)

    deg_kernel = _make_deg_kernel(NW, CH, NP, NC, NS)
    msg_kernel = _make_msg_kernel(NW, CH, NP, NC, NS, H)

    degp = deg_kernel(dst, wpad)
    degp3 = degp.reshape(NC, NP, 1)

    dinv, y1 = pl.pallas_call(
        functools.partial(_tc_a_body, N),
        out_shape=(jax.ShapeDtypeStruct((NP, 1), jnp.float32),
                   jax.ShapeDtypeStruct((N, H), jnp.float32)),
    )(degp3, x, W1)

    ypad = jnp.zeros((NP - N, H), jnp.float32)

    def to_table(y):
        return jnp.concatenate([y, ypad])

    agg1 = msg_kernel(src, dst, wpad, to_table(y1))

    h1, y2 = pl.pallas_call(
        functools.partial(_tc_b_body, N),
        out_shape=(jax.ShapeDtypeStruct((N, H), jnp.float32),
                   jax.ShapeDtypeStruct((N, H), jnp.float32)),
    )(agg1, y1, dinv, b1, bn1_g, bn1_b, W2)

    agg2 = msg_kernel(src, dst, wpad, to_table(y2))

    w1t = l1_wih.T
    b1s = l1_bih + l1_bhh
    Ai, Bi = w1t[:H, 0 * H:1 * H], w1t[H:, 0 * H:1 * H]
    Ag, Bg = w1t[:H, 2 * H:3 * H], w1t[H:, 2 * H:3 * H]
    Ao, Bo = w1t[:H, 3 * H:4 * H], w1t[H:, 3 * H:4 * H]
    w2t = l2_wih.T
    b2s = l2_bih + l2_bhh
    Ci, Cg, Co = w2t[:, 0 * H:1 * H], w2t[:, 2 * H:3 * H], w2t[:, 3 * H:4 * H]
    lwt = lin_W.T
    lwa, lwb, lwc = lwt[:H], lwt[H:2 * H], lwt[2 * H:]

    out = pl.pallas_call(
        functools.partial(_tc_c_body, N),
        out_shape=jax.ShapeDtypeStruct((N, 1), jnp.float32),
    )(agg2, y2, dinv, b2, bn2_g, bn2_b, h1, x,
      Ai, Bi, Ag, Bg, Ao, Bo,
      b1s[0 * H:1 * H], b1s[2 * H:3 * H], b1s[3 * H:4 * H],
      Ci, Cg, Co,
      b2s[0 * H:1 * H], b2s[2 * H:3 * H], b2s[3 * H:4 * H],
      lwa, lwb, lwc, lin_b)
    return out
